# baseline (speedup 1.0000x reference)
import functools

import jax
import jax.numpy as jnp
from jax import lax
from jax.experimental import pallas as pl
from jax.experimental.pallas import tpu as pltpu
from jax.experimental.pallas import tpu_sc as plsc

_NUM_CORES = 2
_NUM_SUBCORES = 16
_NUM_WORKERS = _NUM_CORES * _NUM_SUBCORES
_CHUNK = 128
_NBUF = 4


@functools.partial(jax.jit, static_argnums=(2,))
def _sc_gather_bf16(table_i32, idx, n_chunks_w):
    nbuf = _NBUF
    n_rounds = n_chunks_w // nbuf
    dw = table_i32.shape[1]
    d = 2 * dw

    scratch = [
        pltpu.VMEM((n_chunks_w, _CHUNK), jnp.int32),
        pltpu.VMEM((nbuf, _CHUNK, dw), jnp.int32),
        pltpu.VMEM((nbuf, _CHUNK, d), jnp.float32),
    ]
    scratch += [pltpu.SemaphoreType.DMA] * (2 * nbuf)

    @functools.partial(
        pl.kernel,
        mesh=plsc.VectorSubcoreMesh(core_axis_name="c", subcore_axis_name="s"),
        out_type=jax.ShapeDtypeStruct(
            (_NUM_WORKERS * n_chunks_w, _CHUNK, d), jnp.float32
        ),
        scratch_types=scratch,
        compiler_params=pltpu.CompilerParams(
            use_tc_tiling_on_sc=False, needs_layout_passes=False
        ),
    )
    def body(table_hbm, idx_hbm, out_hbm, idx_v, ri_v, rf_v, *sems):
        gsems = sems[:nbuf]
        ssems = sems[nbuf:]
        wid = lax.axis_index("s") * _NUM_CORES + lax.axis_index("c")
        base = wid * n_chunks_w

        pltpu.sync_copy(idx_hbm.at[wid], idx_v)

        def gather(c, slot):
            return pltpu.make_async_copy(
                table_hbm.at[idx_v.at[c]], ri_v.at[slot], gsems[slot]
            )

        def scatter(c, slot):
            return pltpu.make_async_copy(
                rf_v.at[slot], out_hbm.at[base + c], ssems[slot]
            )

        mask = jnp.int32(-65536)
        sh = jnp.int32(16)

        def expand(slot):
            ri = ri_v.at[slot]
            rf = rf_v.at[slot]

            def row_body(ro, carry):
                for u in range(4):
                    r = ro * 4 + u
                    iv0 = ri[r, pl.ds(0, 16)]
                    iv1 = ri[r, pl.ds(16, 16)]
                    rf[r, pl.ds(0, 16)] = plsc.bitcast(iv0 << sh, jnp.float32)
                    rf[r, pl.ds(16, 16)] = plsc.bitcast(iv0 & mask, jnp.float32)
                    rf[r, pl.ds(32, 16)] = plsc.bitcast(iv1 << sh, jnp.float32)
                    rf[r, pl.ds(48, 16)] = plsc.bitcast(iv1 & mask, jnp.float32)
                return carry

            lax.fori_loop(0, _CHUNK // 4, row_body, 0)

        for slot in range(nbuf):
            gather(slot, slot).start()

        for b in range(nbuf):
            gather(b, b).wait()
            expand(b)
            scatter(b, b).start()
            gather(b + nbuf, b).start()

        def round_body(r, carry):
            c0 = r * nbuf
            for b in range(nbuf):
                c = c0 + b
                gather(c, b).wait()
                scatter(c - nbuf, b).wait()
                expand(b)
                scatter(c, b).start()
                gather(c + nbuf, b).start()
            return carry

        lax.fori_loop(1, n_rounds - 1, round_body, 0)

        c0 = (n_rounds - 1) * nbuf
        for b in range(nbuf):
            c = c0 + b
            gather(c, b).wait()
            scatter(c - nbuf, b).wait()
            expand(b)
            scatter(c, b).start()
        for b in range(nbuf):
            scatter(c0 + b, b).wait()

    return body(table_i32, idx)


def _pack_table(w):
    v, d = w.shape
    tb = w.astype(jnp.bfloat16).reshape(v, 2, 2, d // 4)
    tb = tb.transpose(0, 1, 3, 2)
    return jax.lax.bitcast_convert_type(tb, jnp.int32).reshape(v, d // 2)


def kernel(tokens, embedding_weight):
    bsz, seq = tokens.shape
    _, d = embedding_weight.shape
    n = bsz * seq
    span = _NUM_WORKERS * _CHUNK * _NBUF
    n_pad = -(-n // span) * span
    idx = tokens.astype(jnp.int32).reshape(-1)
    if n_pad != n:
        idx = jnp.pad(idx, (0, n_pad - n))
    n_chunks_w = n_pad // (_NUM_WORKERS * _CHUNK)
    idx = idx.reshape(_NUM_WORKERS, n_chunks_w, _CHUNK)
    out = _sc_gather_bf16(_pack_table(embedding_weight), idx, n_chunks_w)
    out = out.reshape(n_pad, d)[:n]
    return out.reshape(bsz, seq, d)


# speedup vs baseline: 1.0070x; 1.0070x over previous
import functools

import jax
import jax.numpy as jnp
from jax import lax
from jax.experimental import pallas as pl
from jax.experimental.pallas import tpu as pltpu
from jax.experimental.pallas import tpu_sc as plsc

_NUM_CORES = 2
_NUM_SUBCORES = 16
_NUM_WORKERS = _NUM_CORES * _NUM_SUBCORES
_CHUNK = 128
_NBUF = 4


@functools.partial(jax.jit, static_argnums=(2,))
def _sc_gather_bf16(table_i32, idx, n_chunks_w):
    nbuf = _NBUF
    n_rounds = n_chunks_w // nbuf
    dw = table_i32.shape[1]
    d = 2 * dw

    scratch = [
        pltpu.VMEM((n_chunks_w, _CHUNK), jnp.int32),
        pltpu.VMEM((nbuf, _CHUNK, dw), jnp.int32),
        pltpu.VMEM((nbuf, _CHUNK, d), jnp.float32),
    ]
    scratch += [pltpu.SemaphoreType.DMA] * (2 * nbuf)

    @functools.partial(
        pl.kernel,
        mesh=plsc.VectorSubcoreMesh(core_axis_name="c", subcore_axis_name="s"),
        out_type=jax.ShapeDtypeStruct(
            (_NUM_WORKERS * n_chunks_w, _CHUNK, d), jnp.float32
        ),
        scratch_types=scratch,
        compiler_params=pltpu.CompilerParams(
            use_tc_tiling_on_sc=False, needs_layout_passes=False
        ),
    )
    def body(table_hbm, idx_hbm, out_hbm, idx_v, ri_v, rf_v, *sems):
        gsems = sems[:nbuf]
        ssems = sems[nbuf:]
        wid = lax.axis_index("s") * _NUM_CORES + lax.axis_index("c")
        base = wid * n_chunks_w

        pltpu.sync_copy(idx_hbm.at[wid], idx_v)

        def gather(c, slot):
            return pltpu.make_async_copy(
                table_hbm.at[idx_v.at[c]], ri_v.at[slot], gsems[slot]
            )

        def scatter(c, slot):
            return pltpu.make_async_copy(
                rf_v.at[slot], out_hbm.at[base + c], ssems[slot]
            )

        mask = jnp.int32(-65536)
        sh = jnp.int32(16)

        def expand(slot):
            ri = ri_v.at[slot]
            rf = rf_v.at[slot]

            def row_body(ro, carry):
                for u in range(4):
                    r = ro * 4 + u
                    iv0 = ri[r, pl.ds(0, 16)]
                    iv1 = ri[r, pl.ds(16, 16)]
                    rf[r, pl.ds(0, 16)] = plsc.bitcast(iv0 << sh, jnp.float32)
                    rf[r, pl.ds(16, 16)] = plsc.bitcast(iv0 & mask, jnp.float32)
                    rf[r, pl.ds(32, 16)] = plsc.bitcast(iv1 << sh, jnp.float32)
                    rf[r, pl.ds(48, 16)] = plsc.bitcast(iv1 & mask, jnp.float32)
                return carry

            lax.fori_loop(0, _CHUNK // 4, row_body, 0)

        for slot in range(nbuf):
            gather(slot, slot).start()

        for b in range(nbuf):
            gather(b, b).wait()
            expand(b)
            scatter(b, b).start()
            gather(b + nbuf, b).start()

        def round_body(r, carry):
            c0 = r * nbuf
            for b in range(nbuf):
                c = c0 + b
                gather(c, b).wait()
                scatter(c - nbuf, b).wait()
                expand(b)
                scatter(c, b).start()
                gather(c + nbuf, b).start()
            return carry

        lax.fori_loop(1, n_rounds - 1, round_body, 0)

        c0 = (n_rounds - 1) * nbuf
        for b in range(nbuf):
            c = c0 + b
            gather(c, b).wait()
            scatter(c - nbuf, b).wait()
            expand(b)
            scatter(c, b).start()
        for b in range(nbuf):
            scatter(c0 + b, b).wait()

    return body(table_i32, idx)


def _pack_table(w):
    q = w.shape[1] // 4
    u = jax.lax.bitcast_convert_type(w.astype(jnp.bfloat16), jnp.uint16)
    u = u.astype(jnp.int32)
    w0 = u[:, 0 * q:1 * q] | (u[:, 1 * q:2 * q] << 16)
    w1 = u[:, 2 * q:3 * q] | (u[:, 3 * q:4 * q] << 16)
    return jnp.concatenate([w0, w1], axis=1)


def kernel(tokens, embedding_weight):
    bsz, seq = tokens.shape
    _, d = embedding_weight.shape
    n = bsz * seq
    span = _NUM_WORKERS * _CHUNK * _NBUF
    n_pad = -(-n // span) * span
    idx = tokens.astype(jnp.int32).reshape(-1)
    if n_pad != n:
        idx = jnp.pad(idx, (0, n_pad - n))
    n_chunks_w = n_pad // (_NUM_WORKERS * _CHUNK)
    idx = idx.reshape(_NUM_WORKERS, n_chunks_w, _CHUNK)
    out = _sc_gather_bf16(_pack_table(embedding_weight), idx, n_chunks_w)
    out = out.reshape(n_pad, d)[:n]
    return out.reshape(bsz, seq, d)


# speedup vs baseline: 1.0495x; 1.0422x over previous
import functools

import jax
import jax.numpy as jnp
from jax import lax
from jax.experimental import pallas as pl
from jax.experimental.pallas import tpu as pltpu
from jax.experimental.pallas import tpu_sc as plsc

_NUM_CORES = 2
_NUM_SUBCORES = 16
_NUM_WORKERS = _NUM_CORES * _NUM_SUBCORES
_NBUF = 8
_INFLIGHT = 4


@functools.partial(jax.jit, static_argnums=(2, 3, 4))
def _sc_gather(table, idx, bsz, seq, chunk):
    d = table.shape[1]
    hpr = seq // chunk
    n_chunks_w = idx.shape[1]
    nbuf = _NBUF
    lead = _INFLIGHT
    n_rounds = n_chunks_w // nbuf

    scratch = [
        pltpu.VMEM((n_chunks_w, chunk), jnp.int32),
        pltpu.VMEM((nbuf, chunk, d), jnp.float32),
    ]
    scratch += [pltpu.SemaphoreType.DMA] * (2 * nbuf)

    @functools.partial(
        pl.kernel,
        mesh=plsc.VectorSubcoreMesh(core_axis_name="c", subcore_axis_name="s"),
        out_type=jax.ShapeDtypeStruct((bsz, seq, d), jnp.float32),
        scratch_types=scratch,
        compiler_params=pltpu.CompilerParams(use_tc_tiling_on_sc=False),
    )
    def body(table_hbm, idx_hbm, out_hbm, idx_v, rows_v, *sems):
        gsems = sems[:nbuf]
        ssems = sems[nbuf:]
        wid = lax.axis_index("s") * _NUM_CORES + lax.axis_index("c")
        base = wid * n_chunks_w

        pltpu.sync_copy(idx_hbm.at[wid], idx_v)

        def gather(c, slot):
            return pltpu.make_async_copy(
                table_hbm.at[idx_v.at[c]], rows_v.at[slot], gsems[slot]
            )

        def scatter(c, slot):
            cg = base + c
            bi = cg // hpr
            hh = cg - bi * hpr
            return pltpu.make_async_copy(
                rows_v.at[slot],
                out_hbm.at[bi, pl.ds(hh * chunk, chunk)],
                ssems[slot],
            )

        for slot in range(lead):
            gather(slot, slot).start()

        for b in range(nbuf):
            gather(b, b).wait()
            scatter(b, b).start()
            if b >= lead:
                scatter(b - lead, b - lead).wait()
            gather(b + lead, (b + lead) % nbuf).start()

        def round_body(r, carry):
            c0 = r * nbuf
            for b in range(nbuf):
                c = c0 + b
                gather(c, b).wait()
                scatter(c, b).start()
                scatter(c - lead, (b + lead) % nbuf).wait()
                gather(c + lead, (b + lead) % nbuf).start()
            return carry

        lax.fori_loop(1, n_rounds - 1, round_body, 0)

        c0 = (n_rounds - 1) * nbuf
        for b in range(nbuf):
            c = c0 + b
            gather(c, b).wait()
            scatter(c, b).start()
            if b < lead:
                scatter(c - lead, (b + lead) % nbuf).wait()
                gather(c + lead, (b + lead) % nbuf).start()
        for b in range(nbuf):
            scatter(c0 + b, b).wait()

    return body(table, idx)


def kernel(tokens, embedding_weight):
    bsz, seq = tokens.shape
    _, d = embedding_weight.shape
    chunk = 100 if seq % 100 == 0 else seq
    n_chunks = bsz * (seq // chunk)
    idx = tokens.astype(jnp.int32).reshape(_NUM_WORKERS, n_chunks // _NUM_WORKERS, chunk)
    return _sc_gather(embedding_weight, idx, bsz, seq, chunk)
